# Initial kernel scaffold; baseline (speedup 1.0000x reference)
import jax
import jax.numpy as jnp
from jax.experimental import pallas as pl


def kernel(x, edge_index, edge_attr, batch, node_table, edge_table, W1, b1, W2, b2):
    raise NotImplementedError("write your pallas kernel here")


# speedup vs baseline: 3.7294x; 3.7294x over previous
import functools

import jax
import jax.numpy as jnp
from jax import lax
from jax.experimental import pallas as pl
from jax.experimental.pallas import tpu as pltpu
from jax.experimental.pallas import tpu_sc as plsc

_N = 10000
_E = 320000
_DN = 128
_DE = 16
_DM = 256
_NCLS = 128
_NG = 64
_AW = 128
_C = 80
_NC = 2
_NS = 16
_ZR = 125
_RPT = _N // (_NC * _NS) * _NC

_mesh = plsc.VectorSubcoreMesh(core_axis_name="c", subcore_axis_name="s")


def _zero_fill(zb, width):
    def rbody(r, carry):
        for j in range(width // 16):
            zb[r, pl.ds(j * 16, 16)] = jnp.zeros((16,), jnp.float32)
        return carry
    lax.fori_loop(0, _ZR, rbody, 0)


def _zero_acc_slice(acc, zb, s):
    base = s * _RPT
    for k in range(_RPT // _ZR):
        pltpu.sync_copy(zb, acc.at[pl.ds(base + k * _ZR, _ZR), :])


def _dump_acc(acc, out, c, s):
    pltpu.sync_copy(acc.at[pl.ds(s * _RPT, _RPT), :],
                    out.at[pl.ds(c * _N + s * _RPT, _RPT), :])


@functools.partial(
    pl.kernel,
    out_type=(jax.ShapeDtypeStruct((_N, _DN), jnp.float32),
              jax.ShapeDtypeStruct((2 * _N, _AW), jnp.float32)),
    mesh=_mesh,
    scratch_types=[
        pltpu.VMEM_SHARED((_N, _AW), jnp.float32),
        pltpu.VMEM((_C,), jnp.int32),
        pltpu.VMEM((_C,), jnp.int32),
        pltpu.VMEM((_C,), jnp.int32),
        pltpu.VMEM((_C, _DN), jnp.float32),
        pltpu.VMEM((_C, _AW), jnp.float32),
        pltpu.VMEM((_ZR, _AW), jnp.float32),
        pltpu.SemaphoreType.DMA,
        pltpu.SemaphoreType.DMA,
    ],
)
def _sc_prep(x_hbm, dst_hbm, attr_hbm, ntab_hbm, etab_hbm, h_out, eaug_out,
             acc, aidx, didx, xidx, hbuf, ebuf, zb, sem1, sem2):
    c = lax.axis_index("c")
    s = lax.axis_index("s")
    _zero_fill(zb, _AW)
    _zero_acc_slice(acc, zb, s)

    @pl.when(c == 0)
    def _():
        def hbody(i, carry):
            ofs = s * 640 + i * _C
            @pl.when(ofs < _N)
            def _():
                pltpu.sync_copy(x_hbm.at[pl.ds(ofs, _C)], xidx)
                pltpu.async_copy(ntab_hbm.at[xidx], hbuf, sem1).wait()
                pltpu.sync_copy(hbuf, h_out.at[pl.ds(ofs, _C), :])
            return carry
        lax.fori_loop(0, 8, hbody, 0)

    plsc.subcore_barrier()
    ebase = (c * _NS + s) * (_E // (_NC * _NS))
    def ebody(i, carry):
        ofs = ebase + i * _C
        pltpu.sync_copy(attr_hbm.at[pl.ds(ofs, _C)], aidx)
        pltpu.sync_copy(dst_hbm.at[pl.ds(ofs, _C)], didx)
        pltpu.async_copy(etab_hbm.at[aidx], ebuf, sem2).wait()
        pltpu.sync_copy(ebuf, acc.at[didx], add=True)
        return carry
    lax.fori_loop(0, (_E // (_NC * _NS)) // _C, ebody, 0)
    plsc.subcore_barrier()
    _dump_acc(acc, eaug_out, c, s)


@functools.partial(
    pl.kernel,
    out_type=jax.ShapeDtypeStruct((2 * _N, _DN), jnp.float32),
    mesh=_mesh,
    scratch_types=[
        pltpu.VMEM_SHARED((_N, _DN), jnp.float32),
        pltpu.VMEM((_C,), jnp.int32),
        pltpu.VMEM((_C,), jnp.int32),
        pltpu.VMEM((_C, _DN), jnp.float32),
        pltpu.VMEM((_ZR, _DN), jnp.float32),
        pltpu.SemaphoreType.DMA,
    ],
)
def _sc_a1(src_hbm, dst_hbm, h_hbm, out, acc, sidx, didx, rbuf, zb, sem):
    c = lax.axis_index("c")
    s = lax.axis_index("s")
    _zero_fill(zb, _DN)
    _zero_acc_slice(acc, zb, s)
    plsc.subcore_barrier()
    ebase = (c * _NS + s) * (_E // (_NC * _NS))
    def ebody(i, carry):
        ofs = ebase + i * _C
        pltpu.sync_copy(src_hbm.at[pl.ds(ofs, _C)], sidx)
        pltpu.sync_copy(dst_hbm.at[pl.ds(ofs, _C)], didx)
        pltpu.async_copy(h_hbm.at[sidx], rbuf, sem).wait()
        pltpu.sync_copy(rbuf, acc.at[didx], add=True)
        return carry
    lax.fori_loop(0, (_E // (_NC * _NS)) // _C, ebody, 0)
    plsc.subcore_barrier()
    _dump_acc(acc, out, c, s)


@functools.partial(
    pl.kernel,
    out_type=jax.ShapeDtypeStruct((2 * _N, _DN), jnp.float32),
    mesh=_mesh,
    scratch_types=[
        pltpu.VMEM_SHARED((_N, _DN), jnp.float32),
        pltpu.VMEM((_C,), jnp.int32),
        pltpu.VMEM((_C,), jnp.int32),
        pltpu.VMEM((_C, _DN), jnp.float32),
        pltpu.VMEM((_ZR, _DN), jnp.float32),
        pltpu.SemaphoreType.DMA,
    ],
)
def _sc_a2(gidx_hbm, dst_hbm, h1v_hbm, out, acc, sidx, didx, rbuf, zb, sem):
    c = lax.axis_index("c")
    s = lax.axis_index("s")
    _zero_fill(zb, _DN)
    _zero_acc_slice(acc, zb, s)
    plsc.subcore_barrier()
    ebase = s * (_E // _NS)
    def ebody(i, carry):
        ofs = ebase + i * _C
        pltpu.sync_copy(gidx_hbm.at[pl.ds(c * _E + ofs, _C)], sidx)
        pltpu.sync_copy(dst_hbm.at[pl.ds(ofs, _C)], didx)
        pltpu.async_copy(h1v_hbm.at[sidx], rbuf, sem).wait()
        pltpu.sync_copy(rbuf, acc.at[didx], add=True)
        return carry
    lax.fori_loop(0, (_E // _NS) // _C, ebody, 0)
    plsc.subcore_barrier()
    _dump_acc(acc, out, c, s)


_R = 1000


def _tc_layer1_body(h_ref, a1a, a1b, ea, eb, w1h, w1e, b1, emp, o_ref):
    eaug = ea[...] + eb[...]
    es = eaug[:, :_DE] + emp[...]
    denom = eaug[:, _DE:_DE + 1] + 1.0
    acc = jnp.dot(a1a[...] + a1b[...] + h_ref[...], w1h[...],
                  preferred_element_type=jnp.float32)
    acc = acc + jnp.dot(es, w1e[...], preferred_element_type=jnp.float32)
    o_ref[...] = jnp.clip(acc / denom + b1[...], 0.0, 6.0)


def _tc_layer1(h, a1a, a1b, ea, eb, w1h, w1e, b1, emp):
    grid = (_N // _R,)
    return pl.pallas_call(
        _tc_layer1_body,
        grid=grid,
        in_specs=[
            pl.BlockSpec((_R, _DN), lambda i: (i, 0)),
            pl.BlockSpec((_R, _DN), lambda i: (i, 0)),
            pl.BlockSpec((_R, _DN), lambda i: (i, 0)),
            pl.BlockSpec((_R, _AW), lambda i: (i, 0)),
            pl.BlockSpec((_R, _AW), lambda i: (i, 0)),
            pl.BlockSpec((_DN, _DM), lambda i: (0, 0)),
            pl.BlockSpec((_DE, _DM), lambda i: (0, 0)),
            pl.BlockSpec((1, _DM), lambda i: (0, 0)),
            pl.BlockSpec((1, _DE), lambda i: (0, 0)),
        ],
        out_specs=pl.BlockSpec((_R, _DM), lambda i: (i, 0)),
        out_shape=jax.ShapeDtypeStruct((_N, _DM), jnp.float32),
    )(h, a1a, a1b, ea, eb, w1h, w1e, b1, emp)


def _tc_layer2_body(h1_ref, a2a, a2b, ea, eb, bat_ref, w2a, w2b, w2e, b2, emp,
                    o_ref, pooled, counts):
    i = pl.program_id(0)
    eaug = ea[...] + eb[...]
    es = eaug[:, :_DE] + emp[...]
    denom = eaug[:, _DE:_DE + 1] + 1.0
    h1 = h1_ref[...]
    z = jnp.dot(a2a[...] + h1[:, :_DN], w2a[...],
                preferred_element_type=jnp.float32)
    z = z + jnp.dot(a2b[...] + h1[:, _DN:], w2b[...],
                    preferred_element_type=jnp.float32)
    z = z + jnp.dot(es, w2e[...], preferred_element_type=jnp.float32)
    h2 = jnp.maximum(z / denom + b2[...], 0.0)

    g = lax.broadcasted_iota(jnp.int32, (1, _NG), 1)
    onehot = (bat_ref[...] == g).astype(jnp.float32)
    pc = lax.dot_general(onehot, h2, (((0,), (0,)), ((), ())),
                         preferred_element_type=jnp.float32)
    cc = lax.dot_general(onehot, jnp.ones_like(h2), (((0,), (0,)), ((), ())),
                         preferred_element_type=jnp.float32)

    @pl.when(i == 0)
    def _():
        pooled[...] = jnp.zeros_like(pooled)
        counts[...] = jnp.zeros_like(counts)

    pooled[...] += pc
    counts[...] += cc

    @pl.when(i == (_N // _R) - 1)
    def _():
        p = pooled[...] / jnp.maximum(counts[...], 1.0)
        m = jnp.max(p, axis=1, keepdims=True)
        zz = p - m
        o_ref[...] = zz - jnp.log(jnp.sum(jnp.exp(zz), axis=1, keepdims=True))


def _tc_layer2(h1, a2a, a2b, ea, eb, bat, w2a, w2b, w2e, b2, emp):
    grid = (_N // _R,)
    return pl.pallas_call(
        _tc_layer2_body,
        grid=grid,
        in_specs=[
            pl.BlockSpec((_R, _DM), lambda i: (i, 0)),
            pl.BlockSpec((_R, _DN), lambda i: (i, 0)),
            pl.BlockSpec((_R, _DN), lambda i: (i, 0)),
            pl.BlockSpec((_R, _AW), lambda i: (i, 0)),
            pl.BlockSpec((_R, _AW), lambda i: (i, 0)),
            pl.BlockSpec((_R, 1), lambda i: (i, 0)),
            pl.BlockSpec((_DN, _NCLS), lambda i: (0, 0)),
            pl.BlockSpec((_DN, _NCLS), lambda i: (0, 0)),
            pl.BlockSpec((_DE, _NCLS), lambda i: (0, 0)),
            pl.BlockSpec((1, _NCLS), lambda i: (0, 0)),
            pl.BlockSpec((1, _DE), lambda i: (0, 0)),
        ],
        out_specs=pl.BlockSpec((_NG, _NCLS), lambda i: (0, 0)),
        out_shape=jax.ShapeDtypeStruct((_NG, _NCLS), jnp.float32),
        scratch_shapes=[
            pltpu.VMEM((_NG, _NCLS), jnp.float32),
            pltpu.VMEM((_NG, _NCLS), jnp.float32),
        ],
    )(h1, a2a, a2b, ea, eb, bat, w2a, w2b, w2e, b2, emp)


def kernel(x, edge_index, edge_attr, batch, node_table, edge_table, W1, b1, W2, b2):
    x = x.astype(jnp.int32)
    src = edge_index[0].astype(jnp.int32)
    dst = edge_index[1].astype(jnp.int32)
    attr = edge_attr.astype(jnp.int32)
    bat = batch.astype(jnp.int32).reshape(_N, 1)

    evoc = edge_table.shape[0]
    etab_aug = jnp.concatenate(
        [edge_table, jnp.ones((evoc, 1), jnp.float32),
         jnp.zeros((evoc, _AW - _DE - 1), jnp.float32)], axis=1)

    h, eaug2 = _sc_prep(x, dst, attr, node_table, etab_aug)
    a1 = _sc_a1(src, dst, h)

    emp = edge_table[0].reshape(1, _DE)
    h1 = _tc_layer1(h, a1[:_N], a1[_N:], eaug2[:_N], eaug2[_N:],
                    W1[:_DN], W1[_DN:], b1.reshape(1, _DM), emp)

    gidx = jnp.concatenate([src * 2, src * 2 + 1])
    a2 = _sc_a2(gidx, dst, h1.reshape(2 * _N, _DN))

    return _tc_layer2(h1, a2[:_N], a2[_N:], eaug2[:_N], eaug2[_N:], bat,
                      W2[:_DN], W2[_DN:_DM], W2[_DM:], b2.reshape(1, _NCLS), emp)
